# Initial kernel scaffold; baseline (speedup 1.0000x reference)
import jax
import jax.numpy as jnp
from jax.experimental import pallas as pl


def kernel(x, params):
    raise NotImplementedError("write your pallas kernel here")


# speedup vs baseline: 1.3528x; 1.3528x over previous
import functools

import numpy as np
import jax
import jax.numpy as jnp
from jax import lax
from jax.experimental import pallas as pl
from jax.experimental.pallas import tpu as pltpu
from jax.experimental.pallas import tpu_sc as plsc

D1 = 768
D2 = 256
K = 1024
H = 8
NB = 3
DFF = 4 * D2
DK = D2 // H
B = 8
S = 576
N = B * S

def _mm(a, b):
    return jax.lax.dot_general(a, b, (((1,), (0,)), ((), ())),
                               preferred_element_type=jnp.float32)


def _mm_k256(a, b):
    kd = a.shape[1]
    acc = _mm(a[:, 0:256], b[0:256])
    for o in range(256, kd, 256):
        acc = acc + _mm(a[:, o:o + 256], b[o:o + 256])
    return acc


def _erfc(x):
    ax = jnp.abs(x)
    x2 = x * x
    p = jnp.float32(7.85386146e-05)
    for c in (-0.000801019371, 0.00518832775, -0.0268538129, 0.112835854,
              -0.37612626, 1.12837911):
        p = p * x2 + jnp.float32(c)
    one_minus_erf = 1.0 - x * p
    nx2 = -x2
    e_over_ax = jnp.exp(nx2) * (1.0 / ax)
    z = 1.0 / x2
    q1 = jnp.float32(0.0232682)
    for c in (-0.138703942, 0.368742466, -0.582473278, 0.621000469,
              -0.494451523, 0.340488, -0.274112701, 0.563825965):
        q1 = q1 * z + jnp.float32(c)
    q2 = jnp.float32(-10.477664)
    for c in (12.9772, -7.49551868, 2.92101908, -1.01526523, 0.42184633,
              -0.282076746, 0.564189494):
        q2 = q2 * z + jnp.float32(c)
    poly = jnp.where(ax < 2.0, q1, q2)
    r = e_over_ax * poly
    r = jnp.where(nx2 < -88.7228394, 0.0, r)
    r = jnp.where(x < 0.0, 2.0 - r, r)
    return jnp.where(ax < 1.0, one_minus_erf, r)


def _mm_t(a, b):
    return jax.lax.dot_general(a, b, (((1,), (1,)), ((), ())),
                               preferred_element_type=jnp.float32)


def _layernorm(x, g, b, eps=1e-5):
    mu = jnp.mean(x, axis=-1, keepdims=True)
    var = jnp.var(x, axis=-1, keepdims=True)
    return (x - mu) / jnp.sqrt(var + eps) * g + b


def _encoder_body(x_ref, *refs):
    out_ref = refs[-1]
    wrefs = refs[:-1]
    wp = wrefs[0][...]
    bp = wrefs[1][...]
    h = _mm(x_ref[0], wp) + bp
    pos = 2
    for _ in range(NB):
        (wq, bq, wk, bk, wv, bv, wo, bo, g1, be1, w1, bf1, w2, bf2, g2, be2) = [
            wrefs[pos + i][...] for i in range(16)]
        pos += 16
        q = _mm(h, wq) + bq
        k = _mm(h, wk) + bk
        v = _mm(h, wv) + bv
        ctx_parts = []
        for hd in range(H):
            sl = slice(hd * DK, (hd + 1) * DK)
            qh = q[:, sl]
            kh = k[:, sl]
            vh = v[:, sl]
            s = _mm_t(qh, kh) / np.sqrt(DK)
            a = jax.nn.softmax(s, axis=-1)
            ctx_parts.append(_mm(a, vh))
        ctx = jnp.concatenate(ctx_parts, axis=1)
        attn_out = _mm(ctx, wo) + bo
        h = _layernorm(h + attn_out, g1, be1)
        z = _mm(h, w1) + bf1
        ffh = 0.5 * z * _erfc(-z * np.sqrt(0.5).astype(np.float32))
        ff = _mm_k256(ffh, w2) + bf2
        h = _layernorm(h + ff, g2, be2)
    out_ref[0] = _layernorm(h, wrefs[pos][...], wrefs[pos + 1][...])


def _run_encoder(x, flat_w):
    n_w = len(flat_w)
    w_specs = [pl.BlockSpec(w.shape, lambda b, nd=w.ndim: (0,) * nd)
               for w in flat_w]
    return pl.pallas_call(
        _encoder_body,
        grid=(B,),
        in_specs=[pl.BlockSpec((1, S, D1), lambda b: (b, 0, 0))] + w_specs,
        out_specs=pl.BlockSpec((1, S, D2), lambda b: (b, 0, 0)),
        out_shape=jax.ShapeDtypeStruct((B, S, D2), jnp.float32),
        compiler_params=pltpu.CompilerParams(
            dimension_semantics=("arbitrary",)),
    )(x, *flat_w)


def _sc_gather(idx_flat, table):
    mesh = plsc.VectorSubcoreMesh(core_axis_name="c", subcore_axis_name="s")

    @functools.partial(
        pl.kernel, mesh=mesh,
        out_type=jax.ShapeDtypeStruct((N, D2), jnp.float32),
        scratch_types=[
            pltpu.VMEM((72,), jnp.int32),
            pltpu.VMEM((72, D2), jnp.float32),
            pltpu.SemaphoreType.DMA,
        ],
    )
    def k(idx_hbm, table_hbm, out_hbm, idx_v, rows_v, sem):
        wid = lax.axis_index("s") * 2 + lax.axis_index("c")
        for c in range(2):
            base = wid * 144 + c * 72
            pltpu.sync_copy(idx_hbm.at[pl.ds(base, 72)], idx_v)
            pltpu.async_copy(table_hbm.at[idx_v], rows_v, sem).wait()
            pltpu.sync_copy(rows_v, out_hbm.at[pl.ds(base, 72)])

    return k(idx_flat, table)


def _vq_body(flat_ref, et_ref, idx_ref, loss_ref, perp_ref,
             counts_acc, dsum_acc):
    step = pl.program_id(0)
    f = flat_ref[...]
    et = et_ref[...]
    a = jnp.sum(f * f, axis=1, keepdims=True)
    b2 = jnp.sum(et * et, axis=0, keepdims=True)
    m = _mm(f, et)
    dist = (a + b2) - 2.0 * m
    dmin = jnp.min(dist, axis=1, keepdims=True)
    iota = lax.broadcasted_iota(jnp.int32, (S, K), 1)
    cand = jnp.where(dist == dmin, iota, K)
    idxc = jnp.min(cand, axis=1, keepdims=True)
    idx_ref[0] = idxc
    oneh = (iota == idxc).astype(jnp.float32)

    @pl.when(step == 0)
    def _init():
        counts_acc[...] = jnp.zeros_like(counts_acc)
        dsum_acc[0] = 0.0

    counts_acc[...] += jnp.sum(oneh, axis=0, keepdims=True)
    dsum_acc[0] += jnp.sum(dmin)

    @pl.when(step == B - 1)
    def _finalize():
        p = counts_acc[...] / float(N)
        ent = jnp.sum(p * jnp.log(p + 1e-10))
        perp_ref[0, 0] = jnp.exp(-ent)
        t = dsum_acc[0] / float(N * D2)
        loss_ref[0, 0] = t + 0.25 * t


def _run_vq(flat, et):
    return pl.pallas_call(
        _vq_body,
        grid=(B,),
        in_specs=[
            pl.BlockSpec((S, D2), lambda b: (b, 0)),
            pl.BlockSpec((D2, K), lambda b: (0, 0)),
        ],
        out_specs=[
            pl.BlockSpec((1, S, 1), lambda b: (b, 0, 0)),
            pl.BlockSpec(memory_space=pltpu.SMEM),
            pl.BlockSpec(memory_space=pltpu.SMEM),
        ],
        out_shape=[
            jax.ShapeDtypeStruct((B, S, 1), jnp.int32),
            jax.ShapeDtypeStruct((1, 1), jnp.float32),
            jax.ShapeDtypeStruct((1, 1), jnp.float32),
        ],
        scratch_shapes=[
            pltpu.VMEM((1, K), jnp.float32),
            pltpu.SMEM((1,), jnp.float32),
        ],
        compiler_params=pltpu.CompilerParams(
            dimension_semantics=("arbitrary",)),
    )(flat, et)


def kernel(x, params):
    wp, bp = params['proj']
    flat_w = [wp, bp.reshape(1, D2)]
    for blk in params['blocks']:
        wq, bq = blk['wq']
        wk, bk = blk['wk']
        wv, bv = blk['wv']
        wo, bo = blk['wo']
        g1, be1 = blk['ln1']
        g2, be2 = blk['ln2']
        w1, bf1 = blk['ff1']
        w2, bf2 = blk['ff2']
        flat_w += [wq, bq.reshape(1, D2), wk, bk.reshape(1, D2),
                   wv, bv.reshape(1, D2), wo, bo.reshape(1, D2),
                   g1.reshape(1, D2), be1.reshape(1, D2),
                   w1, bf1.reshape(1, DFF), w2, bf2.reshape(1, D2),
                   g2.reshape(1, D2), be2.reshape(1, D2)]
    gp, bpre = params['pre_ln']
    flat_w += [gp.reshape(1, D2), bpre.reshape(1, D2)]

    h = _run_encoder(x, flat_w)
    flat = h.reshape(N, D2)
    et = params['codebook'].T
    idx3, loss, perp = _run_vq(flat, et)
    idx = idx3.reshape(B, S)
    table = params['codebook'].astype(jnp.bfloat16).astype(jnp.float32)
    quant = _sc_gather(idx3.reshape(N), table)
    quant_st = quant.reshape(B, S, D2)
    return quant_st, loss.reshape(()), perp.reshape(()), idx


# speedup vs baseline: 1.3764x; 1.0175x over previous
import functools

import numpy as np
import jax
import jax.numpy as jnp
from jax import lax
from jax.experimental import pallas as pl
from jax.experimental.pallas import tpu as pltpu
from jax.experimental.pallas import tpu_sc as plsc

D1 = 768
D2 = 256
K = 1024
H = 8
NB = 3
DFF = 4 * D2
DK = D2 // H
B = 8
S = 576
N = B * S

def _mm(a, b):
    return jax.lax.dot_general(a, b, (((1,), (0,)), ((), ())),
                               preferred_element_type=jnp.float32)


def _mm_k256(a, b):
    kd = a.shape[1]
    acc = _mm(a[:, 0:256], b[0:256])
    for o in range(256, kd, 256):
        acc = acc + _mm(a[:, o:o + 256], b[o:o + 256])
    return acc


def _erfc(x):
    ax = jnp.abs(x)
    x2 = x * x
    p = jnp.float32(7.85386146e-05)
    for c in (-0.000801019371, 0.00518832775, -0.0268538129, 0.112835854,
              -0.37612626, 1.12837911):
        p = p * x2 + jnp.float32(c)
    one_minus_erf = 1.0 - x * p
    nx2 = -x2
    e_over_ax = jnp.exp(nx2) * (1.0 / ax)
    z = 1.0 / x2
    q1 = jnp.float32(0.0232682)
    for c in (-0.138703942, 0.368742466, -0.582473278, 0.621000469,
              -0.494451523, 0.340488, -0.274112701, 0.563825965):
        q1 = q1 * z + jnp.float32(c)
    q2 = jnp.float32(-10.477664)
    for c in (12.9772, -7.49551868, 2.92101908, -1.01526523, 0.42184633,
              -0.282076746, 0.564189494):
        q2 = q2 * z + jnp.float32(c)
    poly = jnp.where(ax < 2.0, q1, q2)
    r = e_over_ax * poly
    r = jnp.where(nx2 < -88.7228394, 0.0, r)
    r = jnp.where(x < 0.0, 2.0 - r, r)
    return jnp.where(ax < 1.0, one_minus_erf, r)


def _mm_t(a, b):
    return jax.lax.dot_general(a, b, (((1,), (1,)), ((), ())),
                               preferred_element_type=jnp.float32)


def _layernorm(x, g, b, eps=1e-5):
    mu = jnp.mean(x, axis=-1, keepdims=True)
    var = jnp.var(x, axis=-1, keepdims=True)
    return (x - mu) / jnp.sqrt(var + eps) * g + b


def _encoder_body(x_ref, *refs):
    out_ref = refs[-1]
    wrefs = refs[:-1]
    wp = wrefs[0][...]
    bp = wrefs[1][...]
    h = _mm(x_ref[0], wp) + bp
    pos = 2
    for _ in range(NB):
        (wq, bq, wk, bk, wv, bv, wo, bo, g1, be1, w1, bf1, w2, bf2, g2, be2) = [
            wrefs[pos + i][...] for i in range(16)]
        pos += 16
        q = _mm(h, wq) + bq
        k = _mm(h, wk) + bk
        v = _mm(h, wv) + bv
        ctx_parts = []
        for hd in range(H):
            sl = slice(hd * DK, (hd + 1) * DK)
            qh = q[:, sl]
            kh = k[:, sl]
            vh = v[:, sl]
            s = _mm_t(qh, kh) / np.sqrt(DK)
            a = jax.nn.softmax(s, axis=-1)
            ctx_parts.append(_mm(a, vh))
        ctx = jnp.concatenate(ctx_parts, axis=1)
        attn_out = _mm(ctx, wo) + bo
        h = _layernorm(h + attn_out, g1, be1)
        z = _mm(h, w1) + bf1
        ffh = 0.5 * z * _erfc(-z * np.sqrt(0.5).astype(np.float32))
        ff = _mm_k256(ffh, w2) + bf2
        h = _layernorm(h + ff, g2, be2)
    out_ref[0] = _layernorm(h, wrefs[pos][...], wrefs[pos + 1][...])


def _run_encoder(x, flat_w):
    w_specs = [pl.BlockSpec(w.shape, lambda b, nd=w.ndim: (0,) * nd)
               for w in flat_w]
    return pl.pallas_call(
        _encoder_body,
        grid=(B,),
        in_specs=[pl.BlockSpec((1, S, D1), lambda b: (b, 0, 0))] + w_specs,
        out_specs=pl.BlockSpec((1, S, D2), lambda b: (b, 0, 0)),
        out_shape=jax.ShapeDtypeStruct((B, S, D2), jnp.float32),
        compiler_params=pltpu.CompilerParams(
            dimension_semantics=("arbitrary",)),
    )(x, *flat_w)


def _sc_gather(idx_flat, table):
    mesh = plsc.VectorSubcoreMesh(core_axis_name="c", subcore_axis_name="s")

    @functools.partial(
        pl.kernel, mesh=mesh,
        out_type=jax.ShapeDtypeStruct((N, D2), jnp.float32),
        scratch_types=[
            pltpu.VMEM((72,), jnp.int32),
            pltpu.VMEM((72, D2), jnp.float32),
            pltpu.SemaphoreType.DMA,
        ],
    )
    def k(idx_hbm, table_hbm, out_hbm, idx_v, rows_v, sem):
        wid = lax.axis_index("s") * 2 + lax.axis_index("c")
        for c in range(2):
            base = wid * 144 + c * 72
            pltpu.sync_copy(idx_hbm.at[pl.ds(base, 72)], idx_v)
            pltpu.async_copy(table_hbm.at[idx_v], rows_v, sem).wait()
            pltpu.sync_copy(rows_v, out_hbm.at[pl.ds(base, 72)])

    return k(idx_flat, table)


def _vq_body(flat_ref, et_ref, idx_ref, loss_ref, perp_ref,
             counts_acc, dsum_acc):
    step = pl.program_id(0)
    f = flat_ref[...]
    et = et_ref[...]
    a = jnp.sum(f * f, axis=1, keepdims=True)
    b2 = jnp.sum(et * et, axis=0, keepdims=True)
    m = _mm(f, et)
    dist = (a + b2) - 2.0 * m
    dmin = jnp.min(dist, axis=1, keepdims=True)
    iota = lax.broadcasted_iota(jnp.int32, (S, K), 1)
    cand = jnp.where(dist == dmin, iota, K)
    idxc = jnp.min(cand, axis=1, keepdims=True)
    idx_ref[0] = idxc
    oneh = (iota == idxc).astype(jnp.float32)

    @pl.when(step == 0)
    def _init():
        counts_acc[...] = jnp.zeros_like(counts_acc)
        dsum_acc[0] = 0.0

    counts_acc[...] += jnp.sum(oneh, axis=0, keepdims=True)
    dsum_acc[0] += jnp.sum(dmin)

    @pl.when(step == B - 1)
    def _finalize():
        p = counts_acc[...] / float(N)
        ent = jnp.sum(p * jnp.log(p + 1e-10))
        perp_ref[0, 0] = jnp.exp(-ent)
        t = dsum_acc[0] / float(N * D2)
        loss_ref[0, 0] = t + 0.25 * t


def _run_vq(flat, et):
    return pl.pallas_call(
        _vq_body,
        grid=(B,),
        in_specs=[
            pl.BlockSpec((S, D2), lambda b: (b, 0)),
            pl.BlockSpec((D2, K), lambda b: (0, 0)),
        ],
        out_specs=[
            pl.BlockSpec((1, S, 1), lambda b: (b, 0, 0)),
            pl.BlockSpec(memory_space=pltpu.SMEM),
            pl.BlockSpec(memory_space=pltpu.SMEM),
        ],
        out_shape=[
            jax.ShapeDtypeStruct((B, S, 1), jnp.int32),
            jax.ShapeDtypeStruct((1, 1), jnp.float32),
            jax.ShapeDtypeStruct((1, 1), jnp.float32),
        ],
        scratch_shapes=[
            pltpu.VMEM((1, K), jnp.float32),
            pltpu.SMEM((1,), jnp.float32),
        ],
        compiler_params=pltpu.CompilerParams(
            dimension_semantics=("arbitrary",)),
    )(flat, et)


def kernel(x, params):
    wp, bp = params['proj']
    flat_w = [wp, bp.reshape(1, D2)]
    for blk in params['blocks']:
        wq, bq = blk['wq']
        wk, bk = blk['wk']
        wv, bv = blk['wv']
        wo, bo = blk['wo']
        g1, be1 = blk['ln1']
        g2, be2 = blk['ln2']
        w1, bf1 = blk['ff1']
        w2, bf2 = blk['ff2']
        flat_w += [wq, bq.reshape(1, D2), wk, bk.reshape(1, D2),
                   wv, bv.reshape(1, D2), wo, bo.reshape(1, D2),
                   g1.reshape(1, D2), be1.reshape(1, D2),
                   w1, bf1.reshape(1, DFF), w2, bf2.reshape(1, D2),
                   g2.reshape(1, D2), be2.reshape(1, D2)]
    gp, bpre = params['pre_ln']
    flat_w += [gp.reshape(1, D2), bpre.reshape(1, D2)]

    h = _run_encoder(x, flat_w)
    flat = h.reshape(N, D2)
    et = params['codebook'].T
    idx3, loss, perp = _run_vq(flat, et)
    idx = idx3.reshape(B, S)
    table = params['codebook'].astype(jnp.bfloat16).astype(jnp.float32)
    quant = _sc_gather(idx3.reshape(N), table)
    quant_st = quant.reshape(B, S, D2)
    return quant_st, loss.reshape(()), perp.reshape(()), idx
